# baseline (speedup 1.0000x reference)
import jax
import jax.numpy as jnp
from jax import lax
from jax.experimental import pallas as pl
from jax.experimental.pallas import tpu as pltpu
from jax.experimental.pallas import tpu_sc as plsc

_B = 16384
_PATCH_DIM = 32
_HIDDEN = 128
_NPLANE = 3
_CHUNK = 128
_NW = 32
_CPW = 16
_ROWS_PER_PLANE = 512


def _sc_gather_body(table_hbm, idx_hbm, out_hbm, raw_v, idx_v, rows_v, sem):
  wid = lax.axis_index("s") * 2 + lax.axis_index("c")
  rows_per_w = _B // _NW
  pltpu.sync_copy(idx_hbm.at[:, pl.ds(rows_per_w * wid, rows_per_w)], raw_v)

  lane = lax.iota(jnp.int32, 16)
  lane_row = lax.shift_right_logical(lane, 2)
  lane_q = lax.bitwise_and(lane, 3)

  def build(j, carry):
    for v in range(8):
      rows = 32 * j + 4 * v + lane_row
      for c in range(_NPLANE):
        if c < 2:
          cols = 4 * c + lane_q
        else:
          cols = jnp.full((16,), 8, jnp.int32)
        vec = plsc.load_gather(raw_v, [cols, rows])
        idx_v[c, j, pl.ds(16 * v, 16)] = vec
    return carry

  lax.fori_loop(0, _CPW, build, 0, unroll=False)

  def plane(c, carry):
    copies = []
    for j in range(_CPW):
      copies.append(
          pltpu.async_copy(
              table_hbm.at[idx_v.at[c, j]], rows_v.at[j], sem))
    for cp in copies:
      cp.wait()
    pltpu.sync_copy(rows_v, out_hbm.at[c, pl.ds(_CPW * wid, _CPW)])
    return carry

  lax.fori_loop(0, _NPLANE, plane, 0, unroll=False)


def _sc_gather(table, idx_flat):
  mesh = plsc.VectorSubcoreMesh(core_axis_name="c", subcore_axis_name="s")
  return pl.kernel(
      _sc_gather_body,
      out_type=jax.ShapeDtypeStruct(
          (_NPLANE, _ROWS_PER_PLANE, _CHUNK, _PATCH_DIM), jnp.bfloat16),
      mesh=mesh,
      scratch_types=[
          pltpu.VMEM((9, _B // _NW), jnp.int32),
          pltpu.VMEM((_NPLANE, _CPW, _CHUNK), jnp.int32),
          pltpu.VMEM((_CPW, _CHUNK, _PATCH_DIM), jnp.bfloat16),
          pltpu.SemaphoreType.DMA,
      ],
      compiler_params=pltpu.CompilerParams(
          use_tc_tiling_on_sc=False, needs_layout_passes=False),
  )(table, idx_flat)


def _inv_body(x_ref, w1_ref, b1_ref, w2t_ref, b2t_ref, o_ref):
  h = jnp.dot(x_ref[...], w1_ref[...], preferred_element_type=jnp.float32)
  h = jnp.maximum(h + b1_ref[...], 0.0)
  o_ref[...] = lax.dot_general(
      w2t_ref[...], h, (((1,), (1,)), ((), ())),
      preferred_element_type=jnp.float32) + b2t_ref[...]


def _inv_head(x, w1, b1, w2t, b2t, block_rows):
  n, k = x.shape
  m = w2t.shape[0]
  return pl.pallas_call(
      _inv_body,
      grid=(n // block_rows,),
      in_specs=[
          pl.BlockSpec((block_rows, k), lambda i: (i, 0)),
          pl.BlockSpec((k, _HIDDEN), lambda i: (0, 0)),
          pl.BlockSpec((1, _HIDDEN), lambda i: (0, 0)),
          pl.BlockSpec((m, _HIDDEN), lambda i: (0, 0)),
          pl.BlockSpec((m, 1), lambda i: (0, 0)),
      ],
      out_specs=pl.BlockSpec((m, block_rows), lambda i: (0, i)),
      out_shape=jax.ShapeDtypeStruct((m, n), jnp.float32),
  )(x, w1, b1, w2t, b2t)


def _near_body(x_ref, w1_ref, b1_ref, w2t_ref, b2t_ref, o_ref):
  h = jnp.dot(x_ref[0], w1_ref[0], preferred_element_type=jnp.float32)
  h += jnp.dot(x_ref[1], w1_ref[1], preferred_element_type=jnp.float32)
  h += jnp.dot(x_ref[2], w1_ref[2], preferred_element_type=jnp.float32)
  h = jnp.maximum(h + b1_ref[...], 0.0)
  o_ref[...] = lax.dot_general(
      w2t_ref[...], h, (((1,), (1,)), ((), ())),
      preferred_element_type=jnp.float32) + b2t_ref[...]


def _near_head(x, w1p, b1, w2t, b2t, block_rows):
  m = w2t.shape[0]
  return pl.pallas_call(
      _near_body,
      grid=(_B // block_rows,),
      in_specs=[
          pl.BlockSpec((_NPLANE, block_rows, _HIDDEN), lambda i: (0, i, 0)),
          pl.BlockSpec((_NPLANE, _HIDDEN, _HIDDEN), lambda i: (0, 0, 0)),
          pl.BlockSpec((1, _HIDDEN), lambda i: (0, 0)),
          pl.BlockSpec((m, _HIDDEN), lambda i: (0, 0)),
          pl.BlockSpec((m, 1), lambda i: (0, 0)),
      ],
      out_specs=pl.BlockSpec((m, block_rows), lambda i: (0, i)),
      out_shape=jax.ShapeDtypeStruct((m, _B), jnp.float32),
  )(x, w1p, b1, w2t, b2t)


def kernel(agent_indices, z_local, patch_embed, W_near1, b_near1, W_near2,
           b_near2, W_inv1, b_inv1, W_inv2, b_inv2):
  gathered = _sc_gather(patch_embed.astype(jnp.bfloat16), agent_indices.T)
  x = gathered.reshape(_NPLANE, _B, _HIDDEN)

  inv_t = _inv_head(
      z_local, W_inv1, b_inv1.reshape(1, -1), W_inv2.T,
      b_inv2.reshape(-1, 1), block_rows=2048)

  w1p = jnp.concatenate(
      [W_near1, jnp.zeros((_NPLANE * _HIDDEN - W_near1.shape[0], _HIDDEN),
                          jnp.float32)], axis=0).reshape(
                              _NPLANE, _HIDDEN, _HIDDEN).astype(jnp.bfloat16)
  near_t = _near_head(
      x, w1p, b_near1.reshape(1, -1), W_near2.T, b_near2.reshape(-1, 1),
      block_rows=8192)
  return (near_t.T, inv_t.T)


# speedup vs baseline: 1.4863x; 1.4863x over previous
import jax
import jax.numpy as jnp
from jax import lax
from jax.experimental import pallas as pl
from jax.experimental.pallas import tpu as pltpu
from jax.experimental.pallas import tpu_sc as plsc

_B = 16384
_PATCH_DIM = 32
_HIDDEN = 128
_NPLANE = 3
_CHUNK = 128
_NW = 32
_CPW = 16
_ROWS_PER_PLANE = 512


def _sc_gather_body(table_hbm, idx_hbm, out_hbm, raw_v, idx_v, rows_v, sem):
  wid = lax.axis_index("s") * 2 + lax.axis_index("c")
  rows_per_w = _B // _NW
  pltpu.sync_copy(idx_hbm.at[:, pl.ds(rows_per_w * wid, rows_per_w)], raw_v)

  lane = lax.iota(jnp.int32, 16)
  lane_row = lax.shift_right_logical(lane, 2)
  lane_q = lax.bitwise_and(lane, 3)

  def build(j, carry):
    for v in range(8):
      rows = 32 * j + 4 * v + lane_row
      for c in range(_NPLANE):
        if c < 2:
          cols = 4 * c + lane_q
        else:
          cols = jnp.full((16,), 8, jnp.int32)
        vec = plsc.load_gather(raw_v, [cols, rows])
        idx_v[c, j, pl.ds(16 * v, 16)] = vec
    return carry

  lax.fori_loop(0, _CPW, build, 0, unroll=False)

  def plane(c, carry):
    copies = []
    for j in range(_CPW):
      copies.append(
          pltpu.async_copy(
              table_hbm.at[idx_v.at[c, j]], rows_v.at[j], sem))
    for cp in copies:
      cp.wait()
    pltpu.sync_copy(rows_v, out_hbm.at[c, pl.ds(_CPW * wid, _CPW)])
    return carry

  lax.fori_loop(0, _NPLANE, plane, 0, unroll=False)


def _sc_gather(table, idx_flat):
  mesh = plsc.VectorSubcoreMesh(core_axis_name="c", subcore_axis_name="s")
  return pl.kernel(
      _sc_gather_body,
      out_type=jax.ShapeDtypeStruct(
          (_NPLANE, _ROWS_PER_PLANE, _CHUNK, _PATCH_DIM), jnp.float32),
      mesh=mesh,
      scratch_types=[
          pltpu.VMEM((9, _B // _NW), jnp.int32),
          pltpu.VMEM((_NPLANE, _CPW, _CHUNK), jnp.int32),
          pltpu.VMEM((_CPW, _CHUNK, _PATCH_DIM), jnp.float32),
          pltpu.SemaphoreType.DMA,
      ],
      compiler_params=pltpu.CompilerParams(
          use_tc_tiling_on_sc=False, needs_layout_passes=False),
  )(table, idx_flat)


def _inv_body(x_ref, w1_ref, b1_ref, w2t_ref, b2t_ref, o_ref):
  h = jnp.dot(x_ref[...], w1_ref[...], preferred_element_type=jnp.float32)
  h = jnp.maximum(h + b1_ref[...], 0.0)
  o_ref[...] = lax.dot_general(
      w2t_ref[...], h, (((1,), (1,)), ((), ())),
      preferred_element_type=jnp.float32) + b2t_ref[...]


def _inv_head(x, w1, b1, w2t, b2t, block_rows):
  n, k = x.shape
  m = w2t.shape[0]
  return pl.pallas_call(
      _inv_body,
      grid=(n // block_rows,),
      in_specs=[
          pl.BlockSpec((block_rows, k), lambda i: (i, 0)),
          pl.BlockSpec((k, _HIDDEN), lambda i: (0, 0)),
          pl.BlockSpec((1, _HIDDEN), lambda i: (0, 0)),
          pl.BlockSpec((m, _HIDDEN), lambda i: (0, 0)),
          pl.BlockSpec((m, 1), lambda i: (0, 0)),
      ],
      out_specs=pl.BlockSpec((m, block_rows), lambda i: (0, i)),
      out_shape=jax.ShapeDtypeStruct((m, n), jnp.float32),
  )(x, w1, b1, w2t, b2t)


def _near_body(x_ref, w1_ref, b1_ref, w2t_ref, b2t_ref, o_ref):
  h = jnp.dot(x_ref[0], w1_ref[0], preferred_element_type=jnp.float32)
  h += jnp.dot(x_ref[1], w1_ref[1], preferred_element_type=jnp.float32)
  h += jnp.dot(x_ref[2], w1_ref[2], preferred_element_type=jnp.float32)
  h = jnp.maximum(h + b1_ref[...], 0.0)
  o_ref[...] = lax.dot_general(
      w2t_ref[...], h, (((1,), (1,)), ((), ())),
      preferred_element_type=jnp.float32) + b2t_ref[...]


def _near_head(x, w1p, b1, w2t, b2t, block_rows):
  m = w2t.shape[0]
  return pl.pallas_call(
      _near_body,
      grid=(_B // block_rows,),
      in_specs=[
          pl.BlockSpec((_NPLANE, block_rows, _HIDDEN), lambda i: (0, i, 0)),
          pl.BlockSpec((_NPLANE, _HIDDEN, _HIDDEN), lambda i: (0, 0, 0)),
          pl.BlockSpec((1, _HIDDEN), lambda i: (0, 0)),
          pl.BlockSpec((m, _HIDDEN), lambda i: (0, 0)),
          pl.BlockSpec((m, 1), lambda i: (0, 0)),
      ],
      out_specs=pl.BlockSpec((m, block_rows), lambda i: (0, i)),
      out_shape=jax.ShapeDtypeStruct((m, _B), jnp.float32),
  )(x, w1p, b1, w2t, b2t)


def kernel(agent_indices, z_local, patch_embed, W_near1, b_near1, W_near2,
           b_near2, W_inv1, b_inv1, W_inv2, b_inv2):
  gathered = _sc_gather(patch_embed, agent_indices.T)
  x = gathered.reshape(_NPLANE, _B, _HIDDEN)

  inv_t = _inv_head(
      z_local, W_inv1, b_inv1.reshape(1, -1), W_inv2.T,
      b_inv2.reshape(-1, 1), block_rows=2048)

  w1p = jnp.concatenate(
      [W_near1, jnp.zeros((_NPLANE * _HIDDEN - W_near1.shape[0], _HIDDEN),
                          jnp.float32)], axis=0).reshape(_NPLANE, _HIDDEN,
                                                         _HIDDEN)
  near_t = _near_head(
      x, w1p, b_near1.reshape(1, -1), W_near2.T, b_near2.reshape(-1, 1),
      block_rows=8192)
  return (near_t.T, inv_t.T)
